# baseline (speedup 1.0000x reference)
import functools

import jax
import jax.numpy as jnp
from jax import lax
from jax.experimental import pallas as pl
from jax.experimental.pallas import tpu as pltpu
from jax.experimental.pallas import tpu_sc as plsc

B = 128
NPG = 64
E = 262144
D = 16
FULL_E = B * NPG * NPG

NC = 2
NS = 16
LANES = 16

NCHUNK = 16
CH_ROWS = FULL_E // NCHUNK
CH_SHIFT = 15
DUMMY = 256
EPT = E // NS
GROUPS = EPT // LANES
KPC = NCHUNK // NC
CAP = 2048
CAPF = CAP + LANES
BATCH = 128
NB = CAP // BATCH
ZROWS = (CH_ROWS + DUMMY) // NS // 2

_mesh = plsc.VectorSubcoreMesh(core_axis_name="c", subcore_axis_name="s")


@functools.partial(
    pl.kernel,
    out_type=jax.ShapeDtypeStruct((FULL_E, D), jnp.float32),
    mesh=_mesh,
    compiler_params=pltpu.CompilerParams(
        use_tc_tiling_on_sc=False, needs_layout_passes=False),
    scratch_types=[
        pltpu.VMEM((EPT,), jnp.int32),
        pltpu.VMEM((EPT,), jnp.int32),
        *[pltpu.VMEM((CAPF,), jnp.int32) for _ in range(KPC)],
        *[pltpu.VMEM((CAPF,), jnp.int32) for _ in range(KPC)],
        pltpu.VMEM((NB, BATCH), jnp.int32),
        pltpu.VMEM((NB, BATCH), jnp.int32),
        pltpu.VMEM((BATCH, D), jnp.float32),
        pltpu.VMEM((ZROWS, D), jnp.float32),
        pltpu.VMEM_SHARED((CH_ROWS + DUMMY, D), jnp.float32),
        pltpu.SemaphoreType.DMA,
    ],
)
def _sc_scatter_add(ei_hbm, attr_hbm, out_hbm,
                    ubuf, vbuf, *rest):
    elist, rlist = rest[:KPC], rest[KPC:2 * KPC]
    e2d, r2d, rows, zbuf, acc, sem = rest[2 * KPC:]
    c = lax.axis_index("c")
    s = lax.axis_index("s")
    iota = lax.broadcasted_iota(jnp.int32, (LANES,), 0)
    ones16i = jnp.ones((LANES,), jnp.int32)
    zero16i = jnp.zeros((LANES,), jnp.int32)
    zero16f = jnp.zeros((LANES,), jnp.float32)

    def _zb(i, carry):
        zbuf[i, :] = zero16f
        return carry
    lax.fori_loop(0, ZROWS, _zb, 0)

    def _pf(i, carry):
        off = i * LANES
        dspread = CH_ROWS + ((off + iota) & (DUMMY - 1))
        for k in range(KPC):
            rlist[k][pl.ds(off, LANES)] = dspread
            elist[k][pl.ds(off, LANES)] = zero16i
        return carry
    lax.fori_loop(0, CAPF // LANES, _pf, 0)

    ebase = s * EPT
    pltpu.sync_copy(ei_hbm.at[0, pl.ds(ebase, EPT)], ubuf)
    pltpu.sync_copy(ei_hbm.at[1, pl.ds(ebase, EPT)], vbuf)

    def _scan(m, cnts):
        off = m * LANES
        u16 = ubuf[pl.ds(off, LANES)]
        v16 = vbuf[pl.ds(off, LANES)]
        slot = (u16 << 6) | (v16 & 63)
        r = slot & (CH_ROWS - 1)
        chunkv = slot >> CH_SHIFT
        eid = ebase + off + iota
        new = []
        for k in range(KPC):
            cnt = cnts[k]
            mk = chunkv == (k * NC + c)
            mi = jnp.where(mk, ones16i, zero16i)
            pres = plsc.cumsum(mi)
            dest = jnp.where(mk, cnt + pres - 1, CAP + iota)
            plsc.store_scatter(elist[k], [dest], eid)
            plsc.store_scatter(rlist[k], [dest], r)
            new.append(cnt + jnp.sum(mi))
        return tuple(new)
    zero = jnp.zeros((), jnp.int32)
    cnts = lax.fori_loop(0, GROUPS, _scan, (zero,) * KPC)

    zstripe = (CH_ROWS + DUMMY) // NS
    orows = CH_ROWS // NS
    for k in range(KPC):
        chunk_id = k * NC + c
        for z in range(zstripe // ZROWS):
            pltpu.sync_copy(zbuf, acc.at[pl.ds(s * zstripe + z * ZROWS, ZROWS), :])
        plsc.subcore_barrier()
        def _cp(i, carry):
            row = i >> 3
            colg = (i & 7) << 4
            src_off = i << 4
            e2d[row, pl.ds(colg, LANES)] = elist[k][pl.ds(src_off, LANES)]
            r2d[row, pl.ds(colg, LANES)] = rlist[k][pl.ds(src_off, LANES)]
            return carry
        lax.fori_loop(0, CAP // LANES, _cp, 0)
        nb = (cnts[k] + (BATCH - 1)) >> 7
        def _gs(j, carry):
            pltpu.async_copy(attr_hbm.at[e2d.at[j]], rows, sem).wait()
            pltpu.sync_copy(rows, acc.at[r2d.at[j]], add=True)
            return carry
        lax.fori_loop(0, nb, _gs, 0)
        plsc.subcore_barrier()
        pltpu.sync_copy(
            acc.at[pl.ds(s * orows, orows), :],
            out_hbm.at[pl.ds(chunk_id * CH_ROWS + s * orows, orows), :])
        plsc.subcore_barrier()


_FCOLS = 4096


def _full_idx_body(o_ref):
    i = pl.program_id(0)
    col = i * _FCOLS + lax.broadcasted_iota(jnp.int32, (2, _FCOLS), 1)
    rowsel = lax.broadcasted_iota(jnp.int32, (2, _FCOLS), 0)
    src = col >> 6
    dst = ((col >> 12) << 6) | (col & 63)
    o_ref[...] = jnp.where(rowsel == 0, src, dst)


def _full_idx():
    return pl.pallas_call(
        _full_idx_body,
        out_shape=jax.ShapeDtypeStruct((2, FULL_E), jnp.int32),
        grid=(FULL_E // _FCOLS,),
        out_specs=pl.BlockSpec((2, _FCOLS), lambda i: (0, i)),
    )()


def kernel(edge_index, edge_attr, batch_vec):
    out_val = _sc_scatter_add(edge_index, edge_attr)
    full_idx = _full_idx()
    return full_idx, out_val


# speedup vs baseline: 1.0902x; 1.0902x over previous
import functools

import jax
import jax.numpy as jnp
from jax import lax
from jax.experimental import pallas as pl
from jax.experimental.pallas import tpu as pltpu
from jax.experimental.pallas import tpu_sc as plsc

B = 128
NPG = 64
E = 262144
D = 16
FULL_E = B * NPG * NPG

NC = 2
NS = 16
LANES = 16

NCHUNK = 16
CH_ROWS = FULL_E // NCHUNK
CH_SHIFT = 15
DUMMY = 256
EPT = E // NS
GROUPS = EPT // LANES
KPC = NCHUNK // NC
CAP = 2048
CAPF = CAP + LANES
BATCH = 128
NB = CAP // BATCH
ZROWS = (CH_ROWS + DUMMY) // NS // 2

_mesh = plsc.VectorSubcoreMesh(core_axis_name="c", subcore_axis_name="s")


@functools.partial(
    pl.kernel,
    out_type=jax.ShapeDtypeStruct((FULL_E, D), jnp.float32),
    mesh=_mesh,
    compiler_params=pltpu.CompilerParams(
        use_tc_tiling_on_sc=False, needs_layout_passes=False),
    scratch_types=[
        pltpu.VMEM((EPT,), jnp.int32),
        pltpu.VMEM((EPT,), jnp.int32),
        *[pltpu.VMEM((CAPF,), jnp.int32) for _ in range(KPC)],
        *[pltpu.VMEM((CAPF,), jnp.int32) for _ in range(KPC)],
        pltpu.VMEM((NB, BATCH), jnp.int32),
        pltpu.VMEM((NB, BATCH), jnp.int32),
        pltpu.VMEM((BATCH, D), jnp.float32),
        pltpu.VMEM((ZROWS, D), jnp.float32),
        pltpu.VMEM_SHARED((CH_ROWS + DUMMY, D), jnp.float32),
        pltpu.SemaphoreType.DMA,
    ],
)
def _sc_scatter_add(ei_hbm, attr_hbm, out_hbm,
                    ubuf, vbuf, *rest):
    elist, rlist = rest[:KPC], rest[KPC:2 * KPC]
    e2d, r2d, rows, zbuf, acc, sem = rest[2 * KPC:]
    c = lax.axis_index("c")
    s = lax.axis_index("s")
    iota = lax.broadcasted_iota(jnp.int32, (LANES,), 0)
    ones16i = jnp.ones((LANES,), jnp.int32)
    zero16i = jnp.zeros((LANES,), jnp.int32)
    zero16f = jnp.zeros((LANES,), jnp.float32)

    def _zb(i, carry):
        zbuf[i, :] = zero16f
        return carry
    lax.fori_loop(0, ZROWS, _zb, 0)

    def _pf(i, carry):
        off = i * LANES
        dspread = CH_ROWS + ((off + iota) & (DUMMY - 1))
        for k in range(KPC):
            rlist[k][pl.ds(off, LANES)] = dspread
            elist[k][pl.ds(off, LANES)] = zero16i
        return carry
    lax.fori_loop(0, CAPF // LANES, _pf, 0)

    ebase = s * EPT
    pltpu.sync_copy(ei_hbm.at[0, pl.ds(ebase, EPT)], ubuf)
    pltpu.sync_copy(ei_hbm.at[1, pl.ds(ebase, EPT)], vbuf)

    def _scan(m, cnts):
        off = m * LANES
        u16 = ubuf[pl.ds(off, LANES)]
        v16 = vbuf[pl.ds(off, LANES)]
        slot = (u16 << 6) | (v16 & 63)
        r = slot & (CH_ROWS - 1)
        chunkv = slot >> CH_SHIFT
        eid = ebase + off + iota
        new = []
        for k in range(KPC):
            cnt = cnts[k]
            mk = chunkv == (k * NC + c)
            mi = jnp.where(mk, ones16i, zero16i)
            pres = plsc.cumsum(mi)
            dest = jnp.where(mk, cnt + pres - 1, CAP + iota)
            plsc.store_scatter(elist[k], [dest], eid)
            plsc.store_scatter(rlist[k], [dest], r)
            new.append(cnt + jnp.sum(mi))
        return tuple(new)
    zero = jnp.zeros((), jnp.int32)
    cnts = lax.fori_loop(0, 1, _scan, (zero,) * KPC)

    zstripe = (CH_ROWS + DUMMY) // NS
    orows = CH_ROWS // NS
    for k in range(KPC):
        chunk_id = k * NC + c
        for z in range(zstripe // ZROWS):
            pltpu.sync_copy(zbuf, acc.at[pl.ds(s * zstripe + z * ZROWS, ZROWS), :])
        plsc.subcore_barrier()
        def _cp(i, carry):
            row = i >> 3
            colg = (i & 7) << 4
            src_off = i << 4
            e2d[row, pl.ds(colg, LANES)] = elist[k][pl.ds(src_off, LANES)]
            r2d[row, pl.ds(colg, LANES)] = rlist[k][pl.ds(src_off, LANES)]
            return carry
        lax.fori_loop(0, CAP // LANES, _cp, 0)
        nb = (cnts[k] + (BATCH - 1)) >> 7
        def _gs(j, carry):
            pltpu.async_copy(attr_hbm.at[e2d.at[j]], rows, sem).wait()
            pltpu.sync_copy(rows, acc.at[r2d.at[j]], add=True)
            return carry
        lax.fori_loop(0, nb, _gs, 0)
        plsc.subcore_barrier()
        pltpu.sync_copy(
            acc.at[pl.ds(s * orows, orows), :],
            out_hbm.at[pl.ds(chunk_id * CH_ROWS + s * orows, orows), :])
        plsc.subcore_barrier()


_FCOLS = 4096


def _full_idx_body(o_ref):
    i = pl.program_id(0)
    col = i * _FCOLS + lax.broadcasted_iota(jnp.int32, (2, _FCOLS), 1)
    rowsel = lax.broadcasted_iota(jnp.int32, (2, _FCOLS), 0)
    src = col >> 6
    dst = ((col >> 12) << 6) | (col & 63)
    o_ref[...] = jnp.where(rowsel == 0, src, dst)


def _full_idx():
    return pl.pallas_call(
        _full_idx_body,
        out_shape=jax.ShapeDtypeStruct((2, FULL_E), jnp.int32),
        grid=(FULL_E // _FCOLS,),
        out_specs=pl.BlockSpec((2, _FCOLS), lambda i: (0, i)),
    )()


def kernel(edge_index, edge_attr, batch_vec):
    out_val = _sc_scatter_add(edge_index, edge_attr)
    full_idx = _full_idx()
    return full_idx, out_val


# speedup vs baseline: 1.1460x; 1.0511x over previous
import functools

import jax
import jax.numpy as jnp
from jax import lax
from jax.experimental import pallas as pl
from jax.experimental.pallas import tpu as pltpu
from jax.experimental.pallas import tpu_sc as plsc

B = 128
NPG = 64
E = 262144
D = 16
FULL_E = B * NPG * NPG

NC = 2
NS = 16
LANES = 16

NCHUNK = 16
CH_ROWS = FULL_E // NCHUNK
CH_SHIFT = 15
DUMMY = 256
EPT = E // NS
GROUPS = EPT // LANES
KPC = NCHUNK // NC
CAP = 2048
CAPF = CAP + LANES
BATCH = 128
NB = CAP // BATCH
ZROWS = CH_ROWS // NS // 4
SUB = 512
TCPS = SUB // 128
STG = 2 * TCPS * 8 * 128

_mesh = plsc.VectorSubcoreMesh(core_axis_name="c", subcore_axis_name="s")


@functools.partial(
    pl.kernel,
    out_type=jax.ShapeDtypeStruct((FULL_E * D,), jnp.float32),
    mesh=_mesh,
    compiler_params=pltpu.CompilerParams(
        use_tc_tiling_on_sc=False, needs_layout_passes=False),
    scratch_types=[
        pltpu.VMEM((EPT,), jnp.int32),
        pltpu.VMEM((EPT,), jnp.int32),
        *[pltpu.VMEM((CAPF,), jnp.int32) for _ in range(KPC)],
        *[pltpu.VMEM((CAPF,), jnp.int32) for _ in range(KPC)],
        pltpu.VMEM((NB, BATCH), jnp.int32),
        pltpu.VMEM((NB, BATCH), jnp.int32),
        pltpu.VMEM((BATCH, D), jnp.float32),
        pltpu.VMEM((ZROWS, D), jnp.float32),
        pltpu.VMEM((SUB, D), jnp.float32),
        pltpu.VMEM((STG,), jnp.float32),
        pltpu.VMEM_SHARED((CH_ROWS + DUMMY, D), jnp.float32),
        pltpu.SemaphoreType.DMA,
    ],
)
def _sc_scatter_add(ei_hbm, attr_hbm, out_hbm,
                    ubuf, vbuf, *rest):
    elist, rlist = rest[:KPC], rest[KPC:2 * KPC]
    e2d, r2d, rows, zbuf, vstage, stage, acc, sem = rest[2 * KPC:]
    c = lax.axis_index("c")
    s = lax.axis_index("s")
    iota = lax.broadcasted_iota(jnp.int32, (LANES,), 0)
    ones16i = jnp.ones((LANES,), jnp.int32)
    zero16i = jnp.zeros((LANES,), jnp.int32)
    zero16f = jnp.zeros((LANES,), jnp.float32)

    def _zb(i, carry):
        zbuf[i, :] = zero16f
        return carry
    lax.fori_loop(0, ZROWS, _zb, 0)

    def _pf(i, carry):
        off = i * LANES
        dspread = CH_ROWS + ((off + iota) & (DUMMY - 1))
        for k in range(KPC):
            rlist[k][pl.ds(off, LANES)] = dspread
            elist[k][pl.ds(off, LANES)] = zero16i
        return carry
    lax.fori_loop(0, CAPF // LANES, _pf, 0)

    ebase = s * EPT
    pltpu.sync_copy(ei_hbm.at[0, pl.ds(ebase, EPT)], ubuf)
    pltpu.sync_copy(ei_hbm.at[1, pl.ds(ebase, EPT)], vbuf)

    def _scan(m, cnts):
        off = m * LANES
        u16 = ubuf[pl.ds(off, LANES)]
        v16 = vbuf[pl.ds(off, LANES)]
        slot = (u16 << 6) | (v16 & 63)
        r = slot & (CH_ROWS - 1)
        chunkv = slot >> CH_SHIFT
        eid = ebase + off + iota
        new = []
        for k in range(KPC):
            cnt = cnts[k]
            mk = chunkv == (k * NC + c)
            mi = jnp.where(mk, ones16i, zero16i)
            pres = plsc.cumsum(mi)
            dest = jnp.where(mk, cnt + pres - 1, CAP + iota)
            plsc.store_scatter(elist[k], [dest], eid)
            plsc.store_scatter(rlist[k], [dest], r)
            new.append(cnt + jnp.sum(mi))
        return tuple(new)
    zero = jnp.zeros((), jnp.int32)
    cnts = lax.fori_loop(0, GROUPS, _scan, (zero,) * KPC)

    orows = CH_ROWS // NS
    pat = (iota >> 3) * (TCPS * 8 * 128) + (iota & 7) * 128
    for k in range(KPC):
        chunk_id = k * NC + c
        for z in range(orows // ZROWS):
            pltpu.sync_copy(zbuf, acc.at[pl.ds(s * orows + z * ZROWS, ZROWS), :])
        plsc.subcore_barrier()
        def _cp(i, carry):
            row = i >> 3
            colg = (i & 7) << 4
            src_off = i << 4
            e2d[row, pl.ds(colg, LANES)] = elist[k][pl.ds(src_off, LANES)]
            r2d[row, pl.ds(colg, LANES)] = rlist[k][pl.ds(src_off, LANES)]
            return carry
        lax.fori_loop(0, CAP // LANES, _cp, 0)
        nb = (cnts[k] + (BATCH - 1)) >> 7
        def _gs(j, carry):
            pltpu.async_copy(attr_hbm.at[e2d.at[j]], rows, sem).wait()
            pltpu.sync_copy(rows, acc.at[r2d.at[j]], add=True)
            return carry
        lax.fori_loop(0, nb, _gs, 0)
        plsc.subcore_barrier()
        for h in range(orows // SUB):
            pltpu.sync_copy(acc.at[pl.ds(s * orows + h * SUB, SUB), :], vstage)
            def _tp(i, carry):
                row = vstage[i, :]
                scal = (i >> 7) * 1024 + (i & 127)
                plsc.store_scatter(stage, [pat + scal], row)
                return carry
            lax.fori_loop(0, SUB, _tp, 0)
            tc0 = chunk_id * (CH_ROWS // 128) + s * (orows // 128) + h * TCPS
            half = TCPS * 8 * 128
            pltpu.sync_copy(stage.at[pl.ds(0, half)],
                            out_hbm.at[pl.ds(tc0 * 1024, half)])
            pltpu.sync_copy(stage.at[pl.ds(half, half)],
                            out_hbm.at[pl.ds((4096 + tc0) * 1024, half)])
        plsc.subcore_barrier()


_FCOLS = 4096


def _full_idx_body(o_ref):
    i = pl.program_id(0)
    col = i * _FCOLS + lax.broadcasted_iota(jnp.int32, (2, _FCOLS), 1)
    rowsel = lax.broadcasted_iota(jnp.int32, (2, _FCOLS), 0)
    src = col >> 6
    dst = ((col >> 12) << 6) | (col & 63)
    o_ref[...] = jnp.where(rowsel == 0, src, dst)


def _full_idx():
    return pl.pallas_call(
        _full_idx_body,
        out_shape=jax.ShapeDtypeStruct((2, FULL_E), jnp.int32),
        grid=(FULL_E // _FCOLS,),
        out_specs=pl.BlockSpec((2, _FCOLS), lambda i: (0, i)),
    )()


def kernel(edge_index, edge_attr, batch_vec):
    flat = _sc_scatter_add(edge_index, edge_attr)
    out_val = flat.reshape(2, 4096, 8, 128).transpose(0, 2, 1, 3)
    out_val = out_val.reshape(D, FULL_E).T
    full_idx = _full_idx()
    return full_idx, out_val


# speedup vs baseline: 1.2679x; 1.1063x over previous
import functools

import jax
import jax.numpy as jnp
from jax import lax
from jax.experimental import pallas as pl
from jax.experimental.pallas import tpu as pltpu
from jax.experimental.pallas import tpu_sc as plsc

B = 128
NPG = 64
E = 262144
D = 16
FULL_E = B * NPG * NPG

NC = 2
NS = 16
LANES = 16

NCHUNK = 16
CH_ROWS = FULL_E // NCHUNK
CH_SHIFT = 15
DUMMY = 256
EPT = E // NS
GROUPS = EPT // LANES
KPC = NCHUNK // NC
CAP = 2048
CAPF = CAP + LANES
BATCH = 128
NB = CAP // BATCH
ZROWS = CH_ROWS // NS // 4
SUB = 512
TCPS = SUB // 128
STG = 2 * TCPS * 8 * 128

_mesh = plsc.VectorSubcoreMesh(core_axis_name="c", subcore_axis_name="s")


@functools.partial(
    pl.kernel,
    out_type=jax.ShapeDtypeStruct((FULL_E * D,), jnp.float32),
    mesh=_mesh,
    compiler_params=pltpu.CompilerParams(
        use_tc_tiling_on_sc=False, needs_layout_passes=False),
    scratch_types=[
        pltpu.VMEM((EPT,), jnp.int32),
        pltpu.VMEM((EPT,), jnp.int32),
        *[pltpu.VMEM((CAPF,), jnp.int32) for _ in range(KPC)],
        *[pltpu.VMEM((CAPF,), jnp.int32) for _ in range(KPC)],
        pltpu.VMEM((NB, BATCH), jnp.int32),
        pltpu.VMEM((NB, BATCH), jnp.int32),
        pltpu.VMEM((BATCH, D), jnp.float32),
        pltpu.VMEM((ZROWS, D), jnp.float32),
        pltpu.VMEM((SUB, D), jnp.float32),
        pltpu.VMEM((STG,), jnp.float32),
        pltpu.VMEM_SHARED((CH_ROWS + DUMMY, D), jnp.float32),
        pltpu.SemaphoreType.DMA,
    ],
)
def _sc_scatter_add(ei_hbm, attr_hbm, out_hbm,
                    ubuf, vbuf, *rest):
    elist, rlist = rest[:KPC], rest[KPC:2 * KPC]
    e2d, r2d, rows, zbuf, vstage, stage, acc, sem = rest[2 * KPC:]
    c = lax.axis_index("c")
    s = lax.axis_index("s")
    iota = lax.broadcasted_iota(jnp.int32, (LANES,), 0)
    ones16i = jnp.ones((LANES,), jnp.int32)
    zero16i = jnp.zeros((LANES,), jnp.int32)
    zero16f = jnp.zeros((LANES,), jnp.float32)

    @plsc.parallel_loop(0, ZROWS, unroll=8)
    def _zb(i):
        zbuf[i, :] = zero16f

    @plsc.parallel_loop(0, CAPF // LANES, unroll=4)
    def _pf(i):
        off = i * LANES
        dspread = CH_ROWS + ((off + iota) & (DUMMY - 1))
        for k in range(KPC):
            rlist[k][pl.ds(off, LANES)] = dspread
            elist[k][pl.ds(off, LANES)] = zero16i

    ebase = s * EPT
    pltpu.sync_copy(ei_hbm.at[0, pl.ds(ebase, EPT)], ubuf)
    pltpu.sync_copy(ei_hbm.at[1, pl.ds(ebase, EPT)], vbuf)

    def _scan(m, cnts):
        off = m * LANES
        u16 = ubuf[pl.ds(off, LANES)]
        v16 = vbuf[pl.ds(off, LANES)]
        slot = (u16 << 6) | (v16 & 63)
        r = slot & (CH_ROWS - 1)
        chunkv = slot >> CH_SHIFT
        eid = ebase + off + iota
        new = []
        for k in range(KPC):
            cnt = cnts[k]
            mk = chunkv == (k * NC + c)
            mi = jnp.where(mk, ones16i, zero16i)
            pres = plsc.cumsum(mi)
            dest = jnp.where(mk, cnt + pres - 1, CAP + iota)
            plsc.store_scatter(elist[k], [dest], eid)
            plsc.store_scatter(rlist[k], [dest], r)
            new.append(cnt + jnp.sum(mi))
        return tuple(new)
    zero = jnp.zeros((), jnp.int32)
    cnts = lax.fori_loop(0, GROUPS, _scan, (zero,) * KPC)

    orows = CH_ROWS // NS
    pat = (iota >> 3) * (TCPS * 8 * 128) + (iota & 7) * 128
    for k in range(KPC):
        chunk_id = k * NC + c
        for z in range(orows // ZROWS):
            pltpu.sync_copy(zbuf, acc.at[pl.ds(s * orows + z * ZROWS, ZROWS), :])
        plsc.subcore_barrier()
        @plsc.parallel_loop(0, CAP // LANES, unroll=8)
        def _cp(i):
            row = i >> 3
            colg = (i & 7) << 4
            src_off = i << 4
            e2d[row, pl.ds(colg, LANES)] = elist[k][pl.ds(src_off, LANES)]
            r2d[row, pl.ds(colg, LANES)] = rlist[k][pl.ds(src_off, LANES)]
        nb = (cnts[k] + (BATCH - 1)) >> 7
        def _gs(j, carry):
            pltpu.async_copy(attr_hbm.at[e2d.at[j]], rows, sem).wait()
            pltpu.sync_copy(rows, acc.at[r2d.at[j]], add=True)
            return carry
        lax.fori_loop(0, nb, _gs, 0)
        plsc.subcore_barrier()
        for h in range(orows // SUB):
            pltpu.sync_copy(acc.at[pl.ds(s * orows + h * SUB, SUB), :], vstage)
            @plsc.parallel_loop(0, SUB, unroll=8)
            def _tp(i):
                row = vstage[i, :]
                scal = (i >> 7) * 1024 + (i & 127)
                plsc.store_scatter(stage, [pat + scal], row)
            tc0 = chunk_id * (CH_ROWS // 128) + s * (orows // 128) + h * TCPS
            half = TCPS * 8 * 128
            pltpu.sync_copy(stage.at[pl.ds(0, half)],
                            out_hbm.at[pl.ds(tc0 * 1024, half)])
            pltpu.sync_copy(stage.at[pl.ds(half, half)],
                            out_hbm.at[pl.ds((4096 + tc0) * 1024, half)])
        plsc.subcore_barrier()


_FCOLS = 4096


def _full_idx_body(o_ref):
    i = pl.program_id(0)
    col = i * _FCOLS + lax.broadcasted_iota(jnp.int32, (2, _FCOLS), 1)
    rowsel = lax.broadcasted_iota(jnp.int32, (2, _FCOLS), 0)
    src = col >> 6
    dst = ((col >> 12) << 6) | (col & 63)
    o_ref[...] = jnp.where(rowsel == 0, src, dst)


def _full_idx():
    return pl.pallas_call(
        _full_idx_body,
        out_shape=jax.ShapeDtypeStruct((2, FULL_E), jnp.int32),
        grid=(FULL_E // _FCOLS,),
        out_specs=pl.BlockSpec((2, _FCOLS), lambda i: (0, i)),
    )()


def kernel(edge_index, edge_attr, batch_vec):
    flat = _sc_scatter_add(edge_index, edge_attr)
    out_val = flat.reshape(2, 4096, 8, 128).transpose(0, 2, 1, 3)
    out_val = out_val.reshape(D, FULL_E).T
    full_idx = _full_idx()
    return full_idx, out_val


# speedup vs baseline: 1.3037x; 1.0283x over previous
import functools

import jax
import jax.numpy as jnp
from jax import lax
from jax.experimental import pallas as pl
from jax.experimental.pallas import tpu as pltpu
from jax.experimental.pallas import tpu_sc as plsc

B = 128
NPG = 64
E = 262144
D = 16
FULL_E = B * NPG * NPG

NC = 2
NS = 16
LANES = 16

NCHUNK = 16
CH_ROWS = FULL_E // NCHUNK
CH_SHIFT = 15
DUMMY = 64
EPT = E // NS
GROUPS = EPT // LANES
KPC = NCHUNK // NC
CAP = 2048
CAPF = CAP + LANES
BATCH = 128
NB = CAP // BATCH
ZROWS = CH_ROWS // NS // 8
SUB = 512
TCPS = SUB // 128
STG = 2 * TCPS * 8 * 128

_mesh = plsc.VectorSubcoreMesh(core_axis_name="c", subcore_axis_name="s")


@functools.partial(
    pl.kernel,
    out_type=jax.ShapeDtypeStruct((FULL_E * D,), jnp.float32),
    mesh=_mesh,
    compiler_params=pltpu.CompilerParams(
        use_tc_tiling_on_sc=False, needs_layout_passes=False),
    scratch_types=[
        pltpu.VMEM((EPT,), jnp.int32),
        pltpu.VMEM((EPT,), jnp.int32),
        *[pltpu.VMEM((CAPF,), jnp.int32) for _ in range(KPC)],
        *[pltpu.VMEM((CAPF,), jnp.int32) for _ in range(KPC)],
        pltpu.VMEM((NB, BATCH), jnp.int32),
        pltpu.VMEM((NB, BATCH), jnp.int32),
        pltpu.VMEM((BATCH, D), jnp.float32),
        pltpu.VMEM((BATCH, D), jnp.float32),
        pltpu.VMEM((ZROWS, D), jnp.float32),
        pltpu.VMEM((SUB, D), jnp.float32),
        pltpu.VMEM((STG,), jnp.float32),
        pltpu.VMEM_SHARED((CH_ROWS + DUMMY, D), jnp.float32),
        pltpu.SemaphoreType.DMA,
        pltpu.SemaphoreType.DMA,
    ],
)
def _sc_scatter_add(ei_hbm, attr_hbm, out_hbm,
                    ubuf, vbuf, *rest):
    elist, rlist = rest[:KPC], rest[KPC:2 * KPC]
    e2d, r2d, rows0, rows1, zbuf, vstage, stage, acc, sem0, sem1 = rest[2 * KPC:]
    c = lax.axis_index("c")
    s = lax.axis_index("s")
    iota = lax.broadcasted_iota(jnp.int32, (LANES,), 0)
    ones16i = jnp.ones((LANES,), jnp.int32)
    zero16i = jnp.zeros((LANES,), jnp.int32)
    zero16f = jnp.zeros((LANES,), jnp.float32)

    @plsc.parallel_loop(0, ZROWS, unroll=8)
    def _zb(i):
        zbuf[i, :] = zero16f

    @plsc.parallel_loop(0, CAPF // LANES, unroll=4)
    def _pf(i):
        off = i * LANES
        dspread = CH_ROWS + ((off + iota) & (DUMMY - 1))
        for k in range(KPC):
            rlist[k][pl.ds(off, LANES)] = dspread
            elist[k][pl.ds(off, LANES)] = zero16i

    ebase = s * EPT
    pltpu.sync_copy(ei_hbm.at[0, pl.ds(ebase, EPT)], ubuf)
    pltpu.sync_copy(ei_hbm.at[1, pl.ds(ebase, EPT)], vbuf)

    zero = jnp.zeros((), jnp.int32)

    @plsc.parallel_loop(0, GROUPS, unroll=2, carry=(zero,) * KPC)
    def _scan(m, cnts):
        off = m * LANES
        u16 = ubuf[pl.ds(off, LANES)]
        v16 = vbuf[pl.ds(off, LANES)]
        slot = (u16 << 6) | (v16 & 63)
        r = slot & (CH_ROWS - 1)
        chunkv = slot >> CH_SHIFT
        eid = ebase + off + iota
        new = []
        for k in range(KPC):
            cnt = cnts[k]
            mk = chunkv == (k * NC + c)
            mi = jnp.where(mk, ones16i, zero16i)
            pres = plsc.cumsum(mi)
            dest = jnp.where(mk, cnt + pres - 1, CAP + iota)
            plsc.store_scatter(elist[k], [dest], eid)
            plsc.store_scatter(rlist[k], [dest], r)
            new.append(cnt + jnp.sum(mi))
        return tuple(new)
    cnts = _scan

    orows = CH_ROWS // NS
    pat = (iota >> 3) * (TCPS * 8 * 128) + (iota & 7) * 128
    for k in range(KPC):
        chunk_id = k * NC + c
        for z in range(orows // ZROWS):
            pltpu.sync_copy(zbuf, acc.at[pl.ds(s * orows + z * ZROWS, ZROWS), :])
        plsc.subcore_barrier()
        @plsc.parallel_loop(0, CAP // LANES, unroll=8)
        def _cp(i):
            row = i >> 3
            colg = (i & 7) << 4
            src_off = i << 4
            e2d[row, pl.ds(colg, LANES)] = elist[k][pl.ds(src_off, LANES)]
            r2d[row, pl.ds(colg, LANES)] = rlist[k][pl.ds(src_off, LANES)]
        nb = (cnts[k] + (BATCH - 1)) >> 7
        rbufs, gsems = (rows0, rows1), (sem0, sem1)

        @pl.when(nb > 0)
        def _prologue():
            pltpu.async_copy(attr_hbm.at[e2d.at[0]], rows0, sem0)

        def _gs(j, carry):
            for p in (0, 1):
                @pl.when((j & 1) == p)
                def _body():
                    pltpu.make_async_copy(
                        attr_hbm.at[e2d.at[j]], rbufs[p], gsems[p]).wait()

                    @pl.when(j + 1 < nb)
                    def _pref():
                        pltpu.async_copy(
                            attr_hbm.at[e2d.at[j + 1]], rbufs[1 - p],
                            gsems[1 - p])
                    pltpu.sync_copy(rbufs[p], acc.at[r2d.at[j]], add=True)
            return carry
        lax.fori_loop(0, nb, _gs, 0)
        plsc.subcore_barrier()
        for h in range(orows // SUB):
            pltpu.sync_copy(acc.at[pl.ds(s * orows + h * SUB, SUB), :], vstage)
            @plsc.parallel_loop(0, SUB, unroll=8)
            def _tp(i):
                row = vstage[i, :]
                scal = (i >> 7) * 1024 + (i & 127)
                plsc.store_scatter(stage, [pat + scal], row)
            tc0 = chunk_id * (CH_ROWS // 128) + s * (orows // 128) + h * TCPS
            half = TCPS * 8 * 128
            pltpu.sync_copy(stage.at[pl.ds(0, half)],
                            out_hbm.at[pl.ds(tc0 * 1024, half)])
            pltpu.sync_copy(stage.at[pl.ds(half, half)],
                            out_hbm.at[pl.ds((4096 + tc0) * 1024, half)])
        plsc.subcore_barrier()


_FCOLS = 32768


def _full_idx_body(o_ref):
    i = pl.program_id(0)
    col = i * _FCOLS + lax.broadcasted_iota(jnp.int32, (2, _FCOLS), 1)
    rowsel = lax.broadcasted_iota(jnp.int32, (2, _FCOLS), 0)
    src = col >> 6
    dst = ((col >> 12) << 6) | (col & 63)
    o_ref[...] = jnp.where(rowsel == 0, src, dst)


def _full_idx():
    return pl.pallas_call(
        _full_idx_body,
        out_shape=jax.ShapeDtypeStruct((2, FULL_E), jnp.int32),
        grid=(FULL_E // _FCOLS,),
        out_specs=pl.BlockSpec((2, _FCOLS), lambda i: (0, i)),
    )()


def kernel(edge_index, edge_attr, batch_vec):
    flat = _sc_scatter_add(edge_index, edge_attr)
    out_val = flat.reshape(2, 4096, 8, 128).transpose(0, 2, 1, 3)
    out_val = out_val.reshape(D, FULL_E).T
    full_idx = _full_idx()
    return full_idx, out_val


# speedup vs baseline: 1.3292x; 1.0196x over previous
import functools

import jax
import jax.numpy as jnp
from jax import lax
from jax.experimental import pallas as pl
from jax.experimental.pallas import tpu as pltpu
from jax.experimental.pallas import tpu_sc as plsc

B = 128
NPG = 64
E = 262144
D = 16
FULL_E = B * NPG * NPG

NC = 2
NS = 16
LANES = 16

NCHUNK = 16
CH_ROWS = FULL_E // NCHUNK
CH_SHIFT = 15
DUMMY = 64
EPT = E // NS
GROUPS = EPT // LANES
KPC = NCHUNK // NC
CAP = 2048
CAPF = CAP + LANES
BATCH = 128
NB = CAP // BATCH
ZROWS = CH_ROWS // NS // 8
SUB = 512
TCPS = SUB // 128
STG = 2 * TCPS * 8 * 128

_mesh = plsc.VectorSubcoreMesh(core_axis_name="c", subcore_axis_name="s")


@functools.partial(
    pl.kernel,
    out_type=jax.ShapeDtypeStruct((FULL_E * D,), jnp.float32),
    mesh=_mesh,
    compiler_params=pltpu.CompilerParams(
        use_tc_tiling_on_sc=False, needs_layout_passes=False),
    scratch_types=[
        pltpu.VMEM((EPT,), jnp.int32),
        pltpu.VMEM((EPT,), jnp.int32),
        *[pltpu.VMEM((CAPF,), jnp.int32) for _ in range(KPC)],
        *[pltpu.VMEM((CAPF,), jnp.int32) for _ in range(KPC)],
        pltpu.VMEM((NB, BATCH), jnp.int32),
        pltpu.VMEM((NB, BATCH), jnp.int32),
        pltpu.VMEM((BATCH, D), jnp.float32),
        pltpu.VMEM((BATCH, D), jnp.float32),
        pltpu.VMEM((ZROWS, D), jnp.float32),
        pltpu.VMEM((SUB, D), jnp.float32),
        pltpu.VMEM((STG,), jnp.float32),
        pltpu.VMEM_SHARED((CH_ROWS + DUMMY, D), jnp.float32),
        pltpu.SemaphoreType.DMA,
        pltpu.SemaphoreType.DMA,
        pltpu.SemaphoreType.DMA,
        pltpu.SemaphoreType.DMA,
        pltpu.SemaphoreType.DMA,
    ],
)
def _sc_scatter_add(ei_hbm, attr_hbm, out_hbm,
                    ubuf, vbuf, *rest):
    elist, rlist = rest[:KPC], rest[KPC:2 * KPC]
    (e2d, r2d, rows0, rows1, zbuf, vstage, stage, acc,
     sem0, sem1, ssem0, ssem1, osem) = rest[2 * KPC:]
    c = lax.axis_index("c")
    s = lax.axis_index("s")
    iota = lax.broadcasted_iota(jnp.int32, (LANES,), 0)
    ones16i = jnp.ones((LANES,), jnp.int32)
    zero16i = jnp.zeros((LANES,), jnp.int32)
    zero16f = jnp.zeros((LANES,), jnp.float32)

    @plsc.parallel_loop(0, ZROWS, unroll=8)
    def _zb(i):
        zbuf[i, :] = zero16f

    @plsc.parallel_loop(0, CAPF // LANES, unroll=4)
    def _pf(i):
        off = i * LANES
        dspread = CH_ROWS + ((off + iota) & (DUMMY - 1))
        for k in range(KPC):
            rlist[k][pl.ds(off, LANES)] = dspread
            elist[k][pl.ds(off, LANES)] = zero16i

    ebase = s * EPT
    with jax.named_scope("uv_stage"):
        pltpu.sync_copy(ei_hbm.at[0, pl.ds(ebase, EPT)], ubuf)
        pltpu.sync_copy(ei_hbm.at[1, pl.ds(ebase, EPT)], vbuf)

    zero = jnp.zeros((), jnp.int32)
    _scan_scope = jax.named_scope("scan")
    _scan_scope.__enter__()

    @plsc.parallel_loop(0, GROUPS, unroll=2, carry=(zero,) * KPC)
    def _scan(m, cnts):
        off = m * LANES
        u16 = ubuf[pl.ds(off, LANES)]
        v16 = vbuf[pl.ds(off, LANES)]
        slot = (u16 << 6) | (v16 & 63)
        r = slot & (CH_ROWS - 1)
        chunkv = slot >> CH_SHIFT
        eid = ebase + off + iota
        new = []
        for k in range(KPC):
            cnt = cnts[k]
            mk = chunkv == (k * NC + c)
            mi = jnp.where(mk, ones16i, zero16i)
            pres = plsc.cumsum(mi)
            dest = jnp.where(mk, cnt + pres - 1, CAP + iota)
            plsc.store_scatter(elist[k], [dest], eid)
            plsc.store_scatter(rlist[k], [dest], r)
            new.append(cnt + jnp.sum(mi))
        return tuple(new)
    cnts = _scan
    _scan_scope.__exit__(None, None, None)

    orows = CH_ROWS // NS
    pat = (iota >> 3) * (TCPS * 8 * 128) + (iota & 7) * 128
    for k in range(KPC):
        chunk_id = k * NC + c
        with jax.named_scope("zero"):
            for z in range(orows // ZROWS):
                pltpu.sync_copy(zbuf, acc.at[pl.ds(s * orows + z * ZROWS, ZROWS), :])
            plsc.subcore_barrier()
        _cp_scope = jax.named_scope("repack")
        _cp_scope.__enter__()

        @plsc.parallel_loop(0, CAP // LANES, unroll=8)
        def _cp(i):
            row = i >> 3
            colg = (i & 7) << 4
            src_off = i << 4
            e2d[row, pl.ds(colg, LANES)] = elist[k][pl.ds(src_off, LANES)]
            r2d[row, pl.ds(colg, LANES)] = rlist[k][pl.ds(src_off, LANES)]
        _cp_scope.__exit__(None, None, None)
        nb = (cnts[k] + (BATCH - 1)) >> 7
        rbufs, gsems, ssems = (rows0, rows1), (sem0, sem1), (ssem0, ssem1)

        @pl.when(nb > 0)
        def _prologue():
            pltpu.async_copy(attr_hbm.at[e2d.at[0]], rows0, sem0)

        def _gs(j, carry):
            for p in (0, 1):
                @pl.when((j & 1) == p)
                def _body():
                    pltpu.make_async_copy(
                        attr_hbm.at[e2d.at[j]], rbufs[p], gsems[p]).wait()

                    @pl.when(j + 1 < nb)
                    def _pref():
                        pltpu.async_copy(
                            attr_hbm.at[e2d.at[j + 1]], rbufs[1 - p],
                            gsems[1 - p])
                    pltpu.sync_copy(rbufs[p], acc.at[r2d.at[j]], add=True)
            return carry
        with jax.named_scope("gsadd"):
            lax.fori_loop(0, nb, _gs, 0)
            plsc.subcore_barrier()
        _tp_scope = jax.named_scope("tpose")
        _tp_scope.__enter__()
        for h in range(orows // SUB):
            pltpu.sync_copy(acc.at[pl.ds(s * orows + h * SUB, SUB), :], vstage)
            if h > 0:
                pltpu.make_async_copy(
                    stage.at[pl.ds(0, TCPS * 8 * 128)],
                    out_hbm.at[pl.ds(0, TCPS * 8 * 128)], osem).wait()
                pltpu.make_async_copy(
                    stage.at[pl.ds(0, TCPS * 8 * 128)],
                    out_hbm.at[pl.ds(0, TCPS * 8 * 128)], osem).wait()

            @plsc.parallel_loop(0, SUB, unroll=16)
            def _tp(i):
                row = vstage[i, :]
                scal = (i >> 7) * 1024 + (i & 127)
                plsc.store_scatter(stage, [pat + scal], row)
            tc0 = chunk_id * (CH_ROWS // 128) + s * (orows // 128) + h * TCPS
            half = TCPS * 8 * 128
            pltpu.async_copy(stage.at[pl.ds(0, half)],
                             out_hbm.at[pl.ds(tc0 * 1024, half)], osem)
            pltpu.async_copy(stage.at[pl.ds(half, half)],
                             out_hbm.at[pl.ds((4096 + tc0) * 1024, half)], osem)
        pltpu.make_async_copy(
            stage.at[pl.ds(0, TCPS * 8 * 128)],
            out_hbm.at[pl.ds(0, TCPS * 8 * 128)], osem).wait()
        pltpu.make_async_copy(
            stage.at[pl.ds(0, TCPS * 8 * 128)],
            out_hbm.at[pl.ds(0, TCPS * 8 * 128)], osem).wait()
        _tp_scope.__exit__(None, None, None)
        plsc.subcore_barrier()


_FCOLS = 32768


def _full_idx_body(o_ref):
    i = pl.program_id(0)
    col = i * _FCOLS + lax.broadcasted_iota(jnp.int32, (2, _FCOLS), 1)
    rowsel = lax.broadcasted_iota(jnp.int32, (2, _FCOLS), 0)
    src = col >> 6
    dst = ((col >> 12) << 6) | (col & 63)
    o_ref[...] = jnp.where(rowsel == 0, src, dst)


def _full_idx():
    return pl.pallas_call(
        _full_idx_body,
        out_shape=jax.ShapeDtypeStruct((2, FULL_E), jnp.int32),
        grid=(FULL_E // _FCOLS,),
        out_specs=pl.BlockSpec((2, _FCOLS), lambda i: (0, i)),
    )()


def kernel(edge_index, edge_attr, batch_vec):
    flat = _sc_scatter_add(edge_index, edge_attr)
    out_val = flat.reshape(2, 4096, 8, 128).transpose(0, 2, 1, 3)
    out_val = out_val.reshape(D, FULL_E).T
    full_idx = _full_idx()
    return full_idx, out_val


# speedup vs baseline: 1.6671x; 1.2542x over previous
import functools

import jax
import jax.numpy as jnp
from jax import lax
from jax.experimental import pallas as pl
from jax.experimental.pallas import tpu as pltpu
from jax.experimental.pallas import tpu_sc as plsc

B = 128
NPG = 64
E = 262144
D = 16
FULL_E = B * NPG * NPG

NC = 2
NS = 16
LANES = 16

NCHUNK = 16
CH_ROWS = FULL_E // NCHUNK
CH_SHIFT = 15
DUMMY = 64
EPT = E // NS
GROUPS = EPT // LANES
KPC = NCHUNK // NC
CAP = 2048
CAPF = CAP + LANES
BATCH = 128
NB = CAP // BATCH
ZROWS = CH_ROWS // NS // 8
SUB = 512
TCPS = SUB // 128
STG = 2 * TCPS * 8 * 128

_mesh = plsc.VectorSubcoreMesh(core_axis_name="c", subcore_axis_name="s")


@functools.partial(
    pl.kernel,
    out_type=jax.ShapeDtypeStruct((FULL_E * D,), jnp.float32),
    mesh=_mesh,
    compiler_params=pltpu.CompilerParams(
        use_tc_tiling_on_sc=False, needs_layout_passes=False),
    scratch_types=[
        pltpu.VMEM((EPT,), jnp.int32),
        pltpu.VMEM((EPT,), jnp.int32),
        *[pltpu.VMEM((CAPF,), jnp.int32) for _ in range(KPC)],
        *[pltpu.VMEM((CAPF,), jnp.int32) for _ in range(KPC)],
        pltpu.VMEM((NB, BATCH), jnp.int32),
        pltpu.VMEM((NB, BATCH), jnp.int32),
        pltpu.VMEM((BATCH, D), jnp.float32),
        pltpu.VMEM((BATCH, D), jnp.float32),
        pltpu.VMEM((ZROWS, D), jnp.float32),
        pltpu.VMEM((SUB, D), jnp.float32),
        pltpu.VMEM((STG,), jnp.float32),
        pltpu.VMEM_SHARED((CH_ROWS + DUMMY, D), jnp.float32),
        pltpu.SemaphoreType.DMA,
        pltpu.SemaphoreType.DMA,
        pltpu.SemaphoreType.DMA,
        pltpu.SemaphoreType.DMA,
        pltpu.SemaphoreType.DMA,
    ],
)
def _sc_scatter_add(ei_hbm, attr_hbm, out_hbm,
                    ubuf, vbuf, *rest):
    elist, rlist = rest[:KPC], rest[KPC:2 * KPC]
    (e2d, r2d, rows0, rows1, zbuf, vstage, stage, acc,
     sem0, sem1, ssem0, ssem1, osem) = rest[2 * KPC:]
    c = lax.axis_index("c")
    s = lax.axis_index("s")
    iota = lax.broadcasted_iota(jnp.int32, (LANES,), 0)
    ones16i = jnp.ones((LANES,), jnp.int32)
    zero16i = jnp.zeros((LANES,), jnp.int32)
    zero16f = jnp.zeros((LANES,), jnp.float32)

    @plsc.parallel_loop(0, ZROWS, unroll=8)
    def _zb(i):
        zbuf[i, :] = zero16f

    @plsc.parallel_loop(0, CAPF // LANES, unroll=4)
    def _pf(i):
        off = i * LANES
        dspread = CH_ROWS + ((off + iota) & (DUMMY - 1))
        for k in range(KPC):
            rlist[k][pl.ds(off, LANES)] = dspread
            elist[k][pl.ds(off, LANES)] = zero16i

    ebase = s * EPT
    with jax.named_scope("uv_stage"):
        pltpu.sync_copy(ei_hbm.at[0, pl.ds(ebase, EPT)], ubuf)
        pltpu.sync_copy(ei_hbm.at[1, pl.ds(ebase, EPT)], vbuf)

    zero = jnp.zeros((), jnp.int32)
    _scan_scope = jax.named_scope("scan")
    _scan_scope.__enter__()

    @plsc.parallel_loop(0, GROUPS, unroll=2, carry=(zero,) * KPC)
    def _scan(m, cnts):
        off = m * LANES
        u16 = ubuf[pl.ds(off, LANES)]
        v16 = vbuf[pl.ds(off, LANES)]
        slot = (u16 << 6) | (v16 & 63)
        r = slot & (CH_ROWS - 1)
        chunkv = slot >> CH_SHIFT
        eid = ebase + off + iota
        new = []
        for k in range(KPC):
            cnt = cnts[k]
            mk = chunkv == (k * NC + c)
            mi = jnp.where(mk, ones16i, zero16i)
            pres = plsc.cumsum(mi)
            dest = jnp.where(mk, cnt + pres - 1, CAP + iota)
            plsc.store_scatter(elist[k], [dest], eid)
            plsc.store_scatter(rlist[k], [dest], r)
            new.append(cnt + jnp.sum(mi))
        return tuple(new)
    cnts = _scan
    _scan_scope.__exit__(None, None, None)

    orows = CH_ROWS // NS
    pat = (iota >> 3) * (TCPS * 8 * 128) + (iota & 7) * 128
    for k in range(KPC):
        chunk_id = k * NC + c
        with jax.named_scope("zero"):
            for z in range(orows // ZROWS):
                pltpu.sync_copy(zbuf, acc.at[pl.ds(s * orows + z * ZROWS, ZROWS), :])
            plsc.subcore_barrier()
        _cp_scope = jax.named_scope("repack")
        _cp_scope.__enter__()

        @plsc.parallel_loop(0, CAP // LANES, unroll=8)
        def _cp(i):
            row = i >> 3
            colg = (i & 7) << 4
            src_off = i << 4
            e2d[row, pl.ds(colg, LANES)] = elist[k][pl.ds(src_off, LANES)]
            r2d[row, pl.ds(colg, LANES)] = rlist[k][pl.ds(src_off, LANES)]
        _cp_scope.__exit__(None, None, None)
        nb = (cnts[k] + (BATCH - 1)) >> 7
        rbufs, gsems, ssems = (rows0, rows1), (sem0, sem1), (ssem0, ssem1)

        @pl.when(nb > 0)
        def _prologue():
            pltpu.async_copy(attr_hbm.at[e2d.at[0]], rows0, sem0)

        def _gs(j, carry):
            for p in (0, 1):
                @pl.when((j & 1) == p)
                def _body():
                    @pl.when(j + 1 < nb)
                    def _pref():
                        @pl.when(j >= 1)
                        def _drain():
                            pltpu.make_async_copy(
                                rbufs[1 - p], acc.at[r2d.at[j]],
                                ssems[1 - p]).wait()
                        pltpu.async_copy(
                            attr_hbm.at[e2d.at[j + 1]], rbufs[1 - p],
                            gsems[1 - p])
                    pltpu.make_async_copy(
                        attr_hbm.at[e2d.at[j]], rbufs[p], gsems[p]).wait()
                    pltpu.async_copy(rbufs[p], acc.at[r2d.at[j]], ssems[p],
                                     add=True)
            return carry
        with jax.named_scope("gsadd"):
            lax.fori_loop(0, nb, _gs, 0)
            for p in (0, 1):
                @pl.when((nb >= 1) & (((nb - 1) & 1) == p))
                def _final_drain1():
                    pltpu.make_async_copy(
                        rbufs[p], acc.at[r2d.at[0]], ssems[p]).wait()

                @pl.when((nb >= 2) & (((nb - 2) & 1) == p))
                def _final_drain2():
                    pltpu.make_async_copy(
                        rbufs[p], acc.at[r2d.at[0]], ssems[p]).wait()
            plsc.subcore_barrier()
        _tp_scope = jax.named_scope("tpose")
        _tp_scope.__enter__()
        for h in range(orows // SUB):
            pltpu.sync_copy(acc.at[pl.ds(s * orows + h * SUB, SUB), :], vstage)
            if h > 0:
                pltpu.make_async_copy(
                    stage.at[pl.ds(0, TCPS * 8 * 128)],
                    out_hbm.at[pl.ds(0, TCPS * 8 * 128)], osem).wait()
                pltpu.make_async_copy(
                    stage.at[pl.ds(0, TCPS * 8 * 128)],
                    out_hbm.at[pl.ds(0, TCPS * 8 * 128)], osem).wait()

            @plsc.parallel_loop(0, SUB // LANES * D, unroll=16)
            def _tp(i):
                d = ((i >> 8) << 3) | ((i >> 3) & 7)
                slot0 = (((i >> 6) & 3) << 7) | ((i & 7) << 4)
                vals = plsc.load_gather(
                    vstage, [slot0 + iota, jnp.full((LANES,), d, jnp.int32)])
                stage[pl.ds(i << 4, LANES)] = vals
            tc0 = chunk_id * (CH_ROWS // 128) + s * (orows // 128) + h * TCPS
            half = TCPS * 8 * 128
            pltpu.async_copy(stage.at[pl.ds(0, half)],
                             out_hbm.at[pl.ds(tc0 * 1024, half)], osem)
            pltpu.async_copy(stage.at[pl.ds(half, half)],
                             out_hbm.at[pl.ds((4096 + tc0) * 1024, half)], osem)
        pltpu.make_async_copy(
            stage.at[pl.ds(0, TCPS * 8 * 128)],
            out_hbm.at[pl.ds(0, TCPS * 8 * 128)], osem).wait()
        pltpu.make_async_copy(
            stage.at[pl.ds(0, TCPS * 8 * 128)],
            out_hbm.at[pl.ds(0, TCPS * 8 * 128)], osem).wait()
        _tp_scope.__exit__(None, None, None)


_FCOLS = 32768


def _full_idx_body(o_ref):
    i = pl.program_id(0)
    col = i * _FCOLS + lax.broadcasted_iota(jnp.int32, (2, _FCOLS), 1)
    rowsel = lax.broadcasted_iota(jnp.int32, (2, _FCOLS), 0)
    src = col >> 6
    dst = ((col >> 12) << 6) | (col & 63)
    o_ref[...] = jnp.where(rowsel == 0, src, dst)


def _full_idx():
    return pl.pallas_call(
        _full_idx_body,
        out_shape=jax.ShapeDtypeStruct((2, FULL_E), jnp.int32),
        grid=(FULL_E // _FCOLS,),
        out_specs=pl.BlockSpec((2, _FCOLS), lambda i: (0, i)),
    )()


def kernel(edge_index, edge_attr, batch_vec):
    flat = _sc_scatter_add(edge_index, edge_attr)
    out_val = flat.reshape(2, 4096, 8, 128).transpose(0, 2, 1, 3)
    out_val = out_val.reshape(D, FULL_E).T
    full_idx = _full_idx()
    return full_idx, out_val
